# Initial kernel scaffold; baseline (speedup 1.0000x reference)
import jax
import jax.numpy as jnp
from jax.experimental import pallas as pl


def kernel(entity_emb, user_emb, interaction_emb, relation_emb, edge_index, edge_type, interact_indices, interact_values):
    raise NotImplementedError("write your pallas kernel here")


# speedup vs baseline: 1.7121x; 1.7121x over previous
import jax
import jax.numpy as jnp
from jax.experimental import pallas as pl
from jax.experimental.pallas import tpu as pltpu

N_ENTITIES = 50000
N_USERS = 20000
N_RELATIONS = 20
N_INTERACTIONS = 5
DIM = 64


def _s_table_body(e_ref, r_ref, o_ref):
    o_ref[...] = jnp.dot(e_ref[...], r_ref[...],
                         preferred_element_type=jnp.float32) * 0.125


def _s_table(entity_emb, relation_pad_t):
    grid = (N_ENTITIES // 1000,)
    return pl.pallas_call(
        _s_table_body,
        grid=grid,
        in_specs=[
            pl.BlockSpec((1000, DIM), lambda i: (i, 0)),
            pl.BlockSpec((DIM, 128), lambda i: (0, 0)),
        ],
        out_specs=pl.BlockSpec((1000, 128), lambda i: (i, 0)),
        out_shape=jax.ShapeDtypeStruct((N_ENTITIES, 128), jnp.float32),
    )(entity_emb, relation_pad_t)


def kernel(entity_emb, user_emb, interaction_emb, relation_emb,
           edge_index, edge_type, interact_indices, interact_values):
    head = edge_index[0]
    tail = edge_index[1]
    rel_pad = jnp.zeros((DIM, 128), jnp.float32).at[:, :N_RELATIONS].set(relation_emb.T)
    S = _s_table(entity_emb, rel_pad)
    w1 = S[tail, edge_type - 1]
    ex = jnp.exp(w1)
    seg_sum = jax.ops.segment_sum(ex, head, num_segments=N_ENTITIES)
    w = (ex / (seg_sum[head] + 1e-16))[:, None]
    neigh = entity_emb[tail]
    entity_agg = jax.ops.segment_sum(neigh * w, head, num_segments=N_ENTITIES)

    user_index = interact_indices[0]
    item_index = interact_indices[1]
    interact_entity_emb = entity_emb[item_index]
    att_w = jnp.sum(interaction_emb[interact_values] * user_emb[user_index]
                    * interact_entity_emb, axis=1)
    ex2 = jnp.exp(att_w)
    seg2 = jax.ops.segment_sum(ex2, user_index, num_segments=N_USERS)
    w2 = (ex2 / (seg2[user_index] + 1e-16))[:, None]
    user_agg = jax.ops.segment_sum(w2 * interact_entity_emb, user_index,
                                   num_segments=N_USERS)
    return (entity_agg, user_agg)


# speedup vs baseline: 3.8604x; 2.2547x over previous
import functools

import jax
import jax.numpy as jnp
from jax import lax
from jax.experimental import pallas as pl
from jax.experimental.pallas import tpu as pltpu
from jax.experimental.pallas import tpu_sc as plsc

NE = 50000
NU = 20000
NR = 20
NI = 5
E = 800000
NNZ = 500000
D = 64

NC = 2
NS = 16
NW = NC * NS

NEPAD = 50176
NUPAD = 20480
EPAD = 819200
NNZPAD = 524288
HEAD_PAD = 50175
USER_PAD = 20479

C1 = 2560
C2 = 512
C3 = 512
C4 = 512

F32 = jnp.float32
I32 = jnp.int32

_SC_PARAMS = pltpu.CompilerParams(needs_layout_passes=False,
                                  use_tc_tiling_on_sc=False)


def _mesh():
    return plsc.VectorSubcoreMesh(core_axis_name="c", subcore_axis_name="s")


def _wid():
    return lax.axis_index("s") * NC + lax.axis_index("c")


def _zero_1d(ref, n):
    z = jnp.zeros((16,), ref.dtype)

    @pl.loop(0, n // 16)
    def _z(i):
        ref[pl.ds(i * 16, 16)] = z


def _s_table_body(e_ref, r_ref, o_ref):
    o_ref[...] = jnp.dot(e_ref[...], r_ref[...],
                         preferred_element_type=F32) * 0.125


def _s_table(entity_emb, relation_t):
    return pl.pallas_call(
        _s_table_body,
        grid=(NE // 1000,),
        in_specs=[
            pl.BlockSpec((1000, D), lambda i: (i, 0)),
            pl.BlockSpec((D, NR), lambda i: (0, 0)),
        ],
        out_specs=pl.BlockSpec((1000, NR), lambda i: (i, 0)),
        out_shape=jax.ShapeDtypeStruct((NE, NR), F32),
    )(entity_emb, relation_t)


def _ui_table_body(u_ref, i_ref, o_ref):
    for t in range(NI):
        o_ref[t] = u_ref[...] * i_ref[t][None, :]


def _ui_table(user_emb, interaction_emb):
    return pl.pallas_call(
        _ui_table_body,
        grid=(NU // 1000,),
        in_specs=[
            pl.BlockSpec((1000, D), lambda i: (i, 0)),
            pl.BlockSpec((NI, D), lambda i: (0, 0)),
        ],
        out_specs=pl.BlockSpec((NI, 1000, D), lambda i: (0, i, 0)),
        out_shape=jax.ShapeDtypeStruct((NI, NU, D), F32),
    )(user_emb, interaction_emb)


def _k1_body(sflat, head, tail, et, p_all, seg32,
             tail_v, et_v, head_v, idx_v, s_v, p_v, part_v, gsem):
    wid = _wid()
    _zero_1d(part_v, NEPAD)
    per_w = EPAD // NW

    @pl.loop(0, per_w // C1)
    def _chunk(c):
        base = wid * per_w + c * C1
        pltpu.sync_copy(tail.at[pl.ds(base, C1)], tail_v)
        pltpu.sync_copy(et.at[pl.ds(base, C1)], et_v)
        pltpu.sync_copy(head.at[pl.ds(base, C1)], head_v)

        @pl.loop(0, C1 // 16)
        def _g(g):
            sl = pl.ds(g * 16, 16)
            idx_v[sl] = tail_v[sl] * NR + et_v[sl] - 1

        cps = [
            pltpu.async_copy(sflat.at[idx_v.at[pl.ds(j * 128, 128)]],
                             s_v.at[pl.ds(j * 128, 128)], gsem)
            for j in range(C1 // 128)
        ]
        for cp in cps:
            cp.wait()

        @pl.loop(0, C1 // 16)
        def _g2(g):
            sl = pl.ds(g * 16, 16)
            p16 = jnp.exp(s_v[sl])
            p_v[sl] = p16
            plsc.addupdate_scatter(part_v, [head_v[sl]], p16)

        pltpu.sync_copy(p_v, p_all.at[pl.ds(base, C1)])

    pltpu.sync_copy(part_v, seg32.at[wid])


def _k1(sflat, head, tail, et):
    f = pl.kernel(
        _k1_body,
        out_type=(
            jax.ShapeDtypeStruct((EPAD,), F32),
            jax.ShapeDtypeStruct((NW, NEPAD), F32),
        ),
        mesh=_mesh(),
        scratch_types=[
            pltpu.VMEM((C1,), I32),
            pltpu.VMEM((C1,), I32),
            pltpu.VMEM((C1,), I32),
            pltpu.VMEM((C1,), I32),
            pltpu.VMEM((C1,), F32),
            pltpu.VMEM((C1,), F32),
            pltpu.VMEM((NEPAD,), F32),
            pltpu.SemaphoreType.DMA,
        ],
        compiler_params=_SC_PARAMS,
    )
    return f(sflat, head, tail, et)


def _red_body(seg32, segt, acc_v, tmp_v, *, np_):
    wid = _wid()
    st = np_ // NW
    off = wid * st
    _zero_1d(acc_v, st)

    @pl.loop(0, NW)
    def _i(i):
        pltpu.sync_copy(seg32.at[i, pl.ds(off, st)], tmp_v)

        @pl.loop(0, st // 16)
        def _g(g):
            sl = pl.ds(g * 16, 16)
            acc_v[sl] = acc_v[sl] + tmp_v[sl]

    pltpu.sync_copy(acc_v, segt.at[pl.ds(off, st)])


def _reduce_hist(seg32, np_):
    st = np_ // NW
    f = pl.kernel(
        functools.partial(_red_body, np_=np_),
        out_type=jax.ShapeDtypeStruct((np_,), F32),
        mesh=_mesh(),
        scratch_types=[
            pltpu.VMEM((st,), F32),
            pltpu.VMEM((st,), F32),
        ],
        compiler_params=_SC_PARAMS,
    )
    return f(seg32)


def _agg_body(tab, dsti, srci, p_all, segt, out,
              seg_v, head_v, tail_v, p_v, w_v, lh_v, rows_v, acc_sp,
              gsem, ssem, *, rng, outn, pw, c, npass):
    sc = lax.axis_index("c")
    tl = lax.axis_index("s")
    nsub = c // 128
    iota = lax.iota(I32, 16)
    zch = -(-rng // (NS * 128))
    stripe = zch * 128

    for pa in range(npass):
        rid = pa * NC + sc
        pltpu.sync_copy(segt.at[pl.ds(rid * rng, rng)], seg_v)

        @pl.loop(0, 512)
        def _z(g):
            r = lax.div(g, 4)
            cc = lax.rem(g, 4) * 16
            plsc.store_scatter(rows_v, [jnp.full((16,), r), cc + iota],
                               jnp.zeros((16,), F32))

        @pl.loop(0, zch)
        def _zc(k):
            off = jnp.minimum(tl * stripe + k * 128, rng - 128)
            pltpu.sync_copy(rows_v.at[pl.ds(0, 128)],
                            acc_sp.at[pl.ds(off, 128)])

        plsc.subcore_barrier()

        @pl.loop(0, pw // c)
        def _chunk(ch):
            base = tl * pw + ch * c
            pltpu.sync_copy(dsti.at[pl.ds(base, c)], head_v)
            pltpu.sync_copy(srci.at[pl.ds(base, c)], tail_v)
            pltpu.sync_copy(p_all.at[pl.ds(base, c)], p_v)

            @pl.loop(0, c // 16)
            def _g(g):
                sl = pl.ds(g * 16, 16)
                lh = head_v[sl] - rid * rng
                inr = (lh >= 0) & (lh < rng)
                lhc = jnp.clip(lh, 0, rng - 1)
                dn = plsc.load_gather(seg_v, [lhc])
                w = p_v[sl] / (dn + 1e-16)
                w_v[sl] = jnp.where(inr, w, jnp.zeros((16,), F32))
                j = lax.div(g, 8)
                pos = lax.rem(g, 8) * 16
                plsc.store_scatter(lh_v, [jnp.full((16,), j), pos + iota],
                                   lhc)

            cps = [
                pltpu.async_copy(tab.at[tail_v.at[pl.ds(j * 128, 128)]],
                                 rows_v.at[pl.ds(j * 128, 128)], gsem)
                for j in range(nsub)
            ]
            for cp in cps:
                cp.wait()

            @pl.loop(0, c // 16)
            def _s(g):
                for i in range(16):
                    r = g * 16 + i
                    wb = plsc.load_gather(w_v, [jnp.full((16,), r)])
                    ridx = jnp.full((16,), r)
                    for q in range(4):
                        cidx = q * 16 + iota
                        x = plsc.load_gather(rows_v, [ridx, cidx])
                        plsc.store_scatter(rows_v, [ridx, cidx], x * wb)

            sps = [
                pltpu.async_copy(rows_v.at[pl.ds(j * 128, 128)],
                                 acc_sp.at[lh_v.at[j]], ssem, add=True)
                for j in range(nsub)
            ]
            for sp in sps:
                sp.wait()

        plsc.subcore_barrier()
        lim = jnp.minimum(rng - 128, outn - 128 - rid * rng)

        @pl.loop(0, zch)
        def _o(k):
            off = jnp.minimum(tl * stripe + k * 128, lim)
            pltpu.sync_copy(acc_sp.at[pl.ds(off, 128)],
                            out.at[pl.ds(rid * rng + off, 128)])

        if pa + 1 < npass:
            plsc.subcore_barrier()


def _aggregate(tab, dsti, srci, p_all, segt, *, rng, outn, pw, c, npass):
    nsub = c // 128
    f = pl.kernel(
        functools.partial(_agg_body, rng=rng, outn=outn, pw=pw, c=c,
                          npass=npass),
        out_type=jax.ShapeDtypeStruct((outn, D), F32),
        mesh=_mesh(),
        scratch_types=[
            pltpu.VMEM((rng,), F32),
            pltpu.VMEM((c,), I32),
            pltpu.VMEM((c,), I32),
            pltpu.VMEM((c,), F32),
            pltpu.VMEM((c,), F32),
            pltpu.VMEM((nsub, 128), I32),
            pltpu.VMEM((c, D), F32),
            pltpu.VMEM_SHARED((rng, D), F32),
            pltpu.SemaphoreType.DMA,
            pltpu.SemaphoreType.DMA,
        ],
        compiler_params=_SC_PARAMS,
    )
    return f(tab, dsti, srci, p_all, segt)


def _k3_body(ui_tab, ent_tab, user, item, val, p2_all, seg32,
             user_v, item_v, val_v, uix_v, att_v, part_v,
             ui_rows, e_rows, gsem):
    wid = _wid()
    _zero_1d(part_v, NUPAD)
    per_w = NNZPAD // NW
    iota = lax.iota(I32, 16)

    @pl.loop(0, per_w // C3)
    def _chunk(c):
        base = wid * per_w + c * C3
        pltpu.sync_copy(user.at[pl.ds(base, C3)], user_v)
        pltpu.sync_copy(item.at[pl.ds(base, C3)], item_v)
        pltpu.sync_copy(val.at[pl.ds(base, C3)], val_v)

        @pl.loop(0, C3 // 16)
        def _g(g):
            sl = pl.ds(g * 16, 16)
            uix_v[sl] = val_v[sl] * NU + user_v[sl]

        cps = []
        for j in range(C3 // 128):
            cps.append(pltpu.async_copy(
                ui_tab.at[uix_v.at[pl.ds(j * 128, 128)]],
                ui_rows.at[pl.ds(j * 128, 128)], gsem))
            cps.append(pltpu.async_copy(
                ent_tab.at[item_v.at[pl.ds(j * 128, 128)]],
                e_rows.at[pl.ds(j * 128, 128)], gsem))
        for cp in cps:
            cp.wait()

        @pl.loop(0, C3 // 16)
        def _dot(g):
            sl = pl.ds(g * 16, 16)
            ridx = g * 16 + iota
            acc = jnp.zeros((16,), F32)
            for d in range(D):
                cd = jnp.full((16,), d)
                a = plsc.load_gather(ui_rows, [ridx, cd])
                b = plsc.load_gather(e_rows, [ridx, cd])
                acc = acc + a * b
            p16 = jnp.exp(acc)
            att_v[sl] = p16
            plsc.addupdate_scatter(part_v, [user_v[sl]], p16)

        pltpu.sync_copy(att_v, p2_all.at[pl.ds(base, C3)])

    pltpu.sync_copy(part_v, seg32.at[wid])


def _k3(ui_tab, ent_tab, user, item, val):
    f = pl.kernel(
        _k3_body,
        out_type=(
            jax.ShapeDtypeStruct((NNZPAD,), F32),
            jax.ShapeDtypeStruct((NW, NUPAD), F32),
        ),
        mesh=_mesh(),
        scratch_types=[
            pltpu.VMEM((C3,), I32),
            pltpu.VMEM((C3,), I32),
            pltpu.VMEM((C3,), I32),
            pltpu.VMEM((C3,), I32),
            pltpu.VMEM((C3,), F32),
            pltpu.VMEM((NUPAD,), F32),
            pltpu.VMEM((C3, D), F32),
            pltpu.VMEM((C3, D), F32),
            pltpu.SemaphoreType.DMA,
        ],
        compiler_params=_SC_PARAMS,
    )
    return f(ui_tab, ent_tab, user, item, val)


def kernel(entity_emb, user_emb, interaction_emb, relation_emb,
           edge_index, edge_type, interact_indices, interact_values):
    head = edge_index[0]
    tail = edge_index[1]

    S = _s_table(entity_emb, relation_emb.T)
    sflat = S.reshape(NE * NR)
    ui = _ui_table(user_emb, interaction_emb).reshape(NI * NU, D)

    pad_e = EPAD - E
    head_p = jnp.concatenate([head, jnp.full((pad_e,), HEAD_PAD, I32)])
    tail_p = jnp.concatenate([tail, jnp.zeros((pad_e,), I32)])
    et_p = jnp.concatenate([edge_type, jnp.ones((pad_e,), I32)])

    p_all, seg32 = _k1(sflat, head_p, tail_p, et_p)
    segt = _reduce_hist(seg32, NEPAD)
    entity_agg = _aggregate(entity_emb, head_p, tail_p, p_all, segt,
                            rng=12544, outn=NE, pw=EPAD // NS, c=C2,
                            npass=2)

    pad_n = NNZPAD - NNZ
    user_p = jnp.concatenate([interact_indices[0],
                              jnp.full((pad_n,), USER_PAD, I32)])
    item_p = jnp.concatenate([interact_indices[1], jnp.zeros((pad_n,), I32)])
    val_p = jnp.concatenate([interact_values, jnp.zeros((pad_n,), I32)])

    p2_all, seg32u = _k3(ui, entity_emb, user_p, item_p, val_p)
    segu = _reduce_hist(seg32u, NUPAD)
    user_agg = _aggregate(entity_emb, user_p, item_p, p2_all, segu,
                          rng=10240, outn=NU, pw=NNZPAD // NS, c=C4,
                          npass=1)

    return (entity_agg, user_agg)


# speedup vs baseline: 4.2945x; 1.1125x over previous
import functools

import jax
import jax.numpy as jnp
from jax import lax
from jax.experimental import pallas as pl
from jax.experimental.pallas import tpu as pltpu
from jax.experimental.pallas import tpu_sc as plsc

NE = 50000
NU = 20000
NR = 20
NI = 5
E = 800000
NNZ = 500000
D = 64

NC = 2
NS = 16
NW = NC * NS

NEPAD = 50176
NUPAD = 20480
EPAD = 819200
NNZPAD = 524288
HEAD_PAD = 50175
USER_PAD = 20479

C1 = 2560
C2 = 512
C3 = 512
C4 = 512

F32 = jnp.float32
I32 = jnp.int32

_SC_PARAMS = pltpu.CompilerParams(needs_layout_passes=False,
                                  use_tc_tiling_on_sc=False)


def _mesh():
    return plsc.VectorSubcoreMesh(core_axis_name="c", subcore_axis_name="s")


def _wid():
    return lax.axis_index("s") * NC + lax.axis_index("c")


def _zero_1d(ref, n):
    z = jnp.zeros((16,), ref.dtype)

    @pl.loop(0, n // 16)
    def _z(i):
        ref[pl.ds(i * 16, 16)] = z


def _s_table_body(e_ref, r_ref, o_ref):
    o_ref[...] = jnp.dot(e_ref[...], r_ref[...],
                         preferred_element_type=F32) * 0.125


def _s_table(entity_emb, relation_t):
    return pl.pallas_call(
        _s_table_body,
        grid=(NE // 1000,),
        in_specs=[
            pl.BlockSpec((1000, D), lambda i: (i, 0)),
            pl.BlockSpec((D, NR), lambda i: (0, 0)),
        ],
        out_specs=pl.BlockSpec((1000, NR), lambda i: (i, 0)),
        out_shape=jax.ShapeDtypeStruct((NE, NR), F32),
    )(entity_emb, relation_t)


def _ui_table_body(u_ref, i_ref, o_ref):
    for t in range(NI):
        o_ref[t] = u_ref[...] * i_ref[t][None, :]


def _ui_table(user_emb, interaction_emb):
    return pl.pallas_call(
        _ui_table_body,
        grid=(NU // 1000,),
        in_specs=[
            pl.BlockSpec((1000, D), lambda i: (i, 0)),
            pl.BlockSpec((NI, D), lambda i: (0, 0)),
        ],
        out_specs=pl.BlockSpec((NI, 1000, D), lambda i: (0, i, 0)),
        out_shape=jax.ShapeDtypeStruct((NI, NU, D), F32),
    )(user_emb, interaction_emb)


def _k1_body(sflat, head, tail, et, p_all, seg32,
             tail_v, et_v, head_v, idx_v, s_v, p_v, part_v, gsem):
    wid = _wid()
    _zero_1d(part_v, NEPAD)
    per_w = EPAD // NW

    @pl.loop(0, per_w // C1)
    def _chunk(c):
        base = wid * per_w + c * C1
        pltpu.sync_copy(tail.at[pl.ds(base, C1)], tail_v)
        pltpu.sync_copy(et.at[pl.ds(base, C1)], et_v)
        pltpu.sync_copy(head.at[pl.ds(base, C1)], head_v)

        @pl.loop(0, C1 // 16)
        def _g(g):
            sl = pl.ds(g * 16, 16)
            idx_v[sl] = tail_v[sl] * NR + et_v[sl] - 1

        cps = [
            pltpu.async_copy(sflat.at[idx_v.at[pl.ds(j * 128, 128)]],
                             s_v.at[pl.ds(j * 128, 128)], gsem)
            for j in range(C1 // 128)
        ]
        for cp in cps:
            cp.wait()

        @pl.loop(0, C1 // 16)
        def _g2(g):
            sl = pl.ds(g * 16, 16)
            p16 = jnp.exp(s_v[sl])
            p_v[sl] = p16
            plsc.addupdate_scatter(part_v, [head_v[sl]], p16)

        pltpu.sync_copy(p_v, p_all.at[pl.ds(base, C1)])

    pltpu.sync_copy(part_v, seg32.at[wid])


def _k1(sflat, head, tail, et):
    f = pl.kernel(
        _k1_body,
        out_type=(
            jax.ShapeDtypeStruct((EPAD,), F32),
            jax.ShapeDtypeStruct((NW, NEPAD), F32),
        ),
        mesh=_mesh(),
        scratch_types=[
            pltpu.VMEM((C1,), I32),
            pltpu.VMEM((C1,), I32),
            pltpu.VMEM((C1,), I32),
            pltpu.VMEM((C1,), I32),
            pltpu.VMEM((C1,), F32),
            pltpu.VMEM((C1,), F32),
            pltpu.VMEM((NEPAD,), F32),
            pltpu.SemaphoreType.DMA,
        ],
        compiler_params=_SC_PARAMS,
    )
    return f(sflat, head, tail, et)


def _red_body(seg32, segt, acc_v, tmp_v, *, np_):
    wid = _wid()
    st = np_ // NW
    off = wid * st
    _zero_1d(acc_v, st)

    @pl.loop(0, NW)
    def _i(i):
        pltpu.sync_copy(seg32.at[i, pl.ds(off, st)], tmp_v)

        @pl.loop(0, st // 16)
        def _g(g):
            sl = pl.ds(g * 16, 16)
            acc_v[sl] = acc_v[sl] + tmp_v[sl]

    pltpu.sync_copy(acc_v, segt.at[pl.ds(off, st)])


def _reduce_hist(seg32, np_):
    st = np_ // NW
    f = pl.kernel(
        functools.partial(_red_body, np_=np_),
        out_type=jax.ShapeDtypeStruct((np_,), F32),
        mesh=_mesh(),
        scratch_types=[
            pltpu.VMEM((st,), F32),
            pltpu.VMEM((st,), F32),
        ],
        compiler_params=_SC_PARAMS,
    )
    return f(seg32)


def _agg_body(tab, dsti, srci, p_all, segt, out,
              seg_v, head_v, tail_v, p_v, w_v, lh_v, rows_v, acc_sp,
              gsem, ssem, *, rng, outn, pw, c, npass):
    sc = lax.axis_index("c")
    tl = lax.axis_index("s")
    nsub = c // 128
    iota = lax.iota(I32, 16)
    zch = -(-rng // (NS * 128))
    stripe = zch * 128

    for pa in range(npass):
        rid = pa * NC + sc
        pltpu.sync_copy(segt.at[pl.ds(rid * rng, rng)], seg_v)

        @pl.loop(0, 512)
        def _z(g):
            r = lax.div(g, 4)
            cc = lax.rem(g, 4) * 16
            plsc.store_scatter(rows_v, [jnp.full((16,), r), cc + iota],
                               jnp.zeros((16,), F32))

        @pl.loop(0, zch)
        def _zc(k):
            off = jnp.minimum(tl * stripe + k * 128, rng - 128)
            pltpu.sync_copy(rows_v.at[pl.ds(0, 128)],
                            acc_sp.at[pl.ds(off, 128)])

        plsc.subcore_barrier()

        @pl.loop(0, pw // c)
        def _chunk(ch):
            base = tl * pw + ch * c
            pltpu.sync_copy(dsti.at[pl.ds(base, c)], head_v)
            pltpu.sync_copy(srci.at[pl.ds(base, c)], tail_v)
            pltpu.sync_copy(p_all.at[pl.ds(base, c)], p_v)

            @pl.loop(0, c // 16)
            def _g(g):
                sl = pl.ds(g * 16, 16)
                lh = head_v[sl] - rid * rng
                inr = (lh >= 0) & (lh < rng)
                lhc = jnp.clip(lh, 0, rng - 1)
                dn = plsc.load_gather(seg_v, [lhc])
                w = p_v[sl] / (dn + 1e-16)
                w_v[sl] = jnp.where(inr, w, jnp.zeros((16,), F32))
                j = lax.div(g, 8)
                pos = lax.rem(g, 8) * 16
                plsc.store_scatter(lh_v, [jnp.full((16,), j), pos + iota],
                                   lhc)

            cps = [
                pltpu.async_copy(tab.at[tail_v.at[pl.ds(j * 128, 128)]],
                                 rows_v.at[pl.ds(j * 128, 128)], gsem)
                for j in range(nsub)
            ]
            for cp in cps:
                cp.wait()

            @pl.loop(0, c // 16)
            def _s(g):
                w16 = w_v[pl.ds(g * 16, 16)]
                for i in range(16):
                    r = g * 16 + i
                    wb = lax.broadcast(w16[i], (16,))
                    for q in range(4):
                        sl2 = pl.ds(q * 16, 16)
                        rows_v[r, sl2] = rows_v[r, sl2] * wb

            sps = [
                pltpu.async_copy(rows_v.at[pl.ds(j * 128, 128)],
                                 acc_sp.at[lh_v.at[j]], ssem, add=True)
                for j in range(nsub)
            ]
            for sp in sps:
                sp.wait()

        plsc.subcore_barrier()
        lim = jnp.minimum(rng - 128, outn - 128 - rid * rng)

        @pl.loop(0, zch)
        def _o(k):
            off = jnp.minimum(tl * stripe + k * 128, lim)
            pltpu.sync_copy(acc_sp.at[pl.ds(off, 128)],
                            out.at[pl.ds(rid * rng + off, 128)])

        if pa + 1 < npass:
            plsc.subcore_barrier()


def _aggregate(tab, dsti, srci, p_all, segt, *, rng, outn, pw, c, npass):
    nsub = c // 128
    f = pl.kernel(
        functools.partial(_agg_body, rng=rng, outn=outn, pw=pw, c=c,
                          npass=npass),
        out_type=jax.ShapeDtypeStruct((outn, D), F32),
        mesh=_mesh(),
        scratch_types=[
            pltpu.VMEM((rng,), F32),
            pltpu.VMEM((c,), I32),
            pltpu.VMEM((c,), I32),
            pltpu.VMEM((c,), F32),
            pltpu.VMEM((c,), F32),
            pltpu.VMEM((nsub, 128), I32),
            pltpu.VMEM((c, D), F32),
            pltpu.VMEM_SHARED((rng, D), F32),
            pltpu.SemaphoreType.DMA,
            pltpu.SemaphoreType.DMA,
        ],
        compiler_params=_SC_PARAMS,
    )
    return f(tab, dsti, srci, p_all, segt)


def _k3_body(ui_tab, ent_tab, user, item, val, p2_all, seg32,
             user_v, item_v, val_v, uix_v, att_v, part_v,
             ui_rows, e_rows, gsem):
    wid = _wid()
    _zero_1d(part_v, NUPAD)
    per_w = NNZPAD // NW
    iota = lax.iota(I32, 16)

    @pl.loop(0, per_w // C3)
    def _chunk(c):
        base = wid * per_w + c * C3
        pltpu.sync_copy(user.at[pl.ds(base, C3)], user_v)
        pltpu.sync_copy(item.at[pl.ds(base, C3)], item_v)
        pltpu.sync_copy(val.at[pl.ds(base, C3)], val_v)

        @pl.loop(0, C3 // 16)
        def _g(g):
            sl = pl.ds(g * 16, 16)
            uix_v[sl] = val_v[sl] * NU + user_v[sl]

        cps = []
        for j in range(C3 // 128):
            cps.append(pltpu.async_copy(
                ui_tab.at[uix_v.at[pl.ds(j * 128, 128)]],
                ui_rows.at[pl.ds(j * 128, 128)], gsem))
            cps.append(pltpu.async_copy(
                ent_tab.at[item_v.at[pl.ds(j * 128, 128)]],
                e_rows.at[pl.ds(j * 128, 128)], gsem))
        for cp in cps:
            cp.wait()

        @pl.loop(0, C3 // 16)
        def _dot(g):
            sl = pl.ds(g * 16, 16)
            ridx = g * 16 + iota
            accs = [jnp.zeros((16,), F32) for _ in range(4)]
            for d0 in range(0, D, 4):
                for k in range(4):
                    cd = jnp.full((16,), d0 + k)
                    a = plsc.load_gather(ui_rows, [ridx, cd])
                    b = plsc.load_gather(e_rows, [ridx, cd])
                    accs[k] = accs[k] + a * b
            p16 = jnp.exp((accs[0] + accs[1]) + (accs[2] + accs[3]))
            att_v[sl] = p16
            plsc.addupdate_scatter(part_v, [user_v[sl]], p16)

        pltpu.sync_copy(att_v, p2_all.at[pl.ds(base, C3)])

    pltpu.sync_copy(part_v, seg32.at[wid])


def _k3(ui_tab, ent_tab, user, item, val):
    f = pl.kernel(
        _k3_body,
        out_type=(
            jax.ShapeDtypeStruct((NNZPAD,), F32),
            jax.ShapeDtypeStruct((NW, NUPAD), F32),
        ),
        mesh=_mesh(),
        scratch_types=[
            pltpu.VMEM((C3,), I32),
            pltpu.VMEM((C3,), I32),
            pltpu.VMEM((C3,), I32),
            pltpu.VMEM((C3,), I32),
            pltpu.VMEM((C3,), F32),
            pltpu.VMEM((NUPAD,), F32),
            pltpu.VMEM((C3, D), F32),
            pltpu.VMEM((C3, D), F32),
            pltpu.SemaphoreType.DMA,
        ],
        compiler_params=_SC_PARAMS,
    )
    return f(ui_tab, ent_tab, user, item, val)


def kernel(entity_emb, user_emb, interaction_emb, relation_emb,
           edge_index, edge_type, interact_indices, interact_values):
    head = edge_index[0]
    tail = edge_index[1]

    S = _s_table(entity_emb, relation_emb.T)
    sflat = S.reshape(NE * NR)
    ui = _ui_table(user_emb, interaction_emb).reshape(NI * NU, D)

    pad_e = EPAD - E
    head_p = jnp.concatenate([head, jnp.full((pad_e,), HEAD_PAD, I32)])
    tail_p = jnp.concatenate([tail, jnp.zeros((pad_e,), I32)])
    et_p = jnp.concatenate([edge_type, jnp.ones((pad_e,), I32)])

    p_all, seg32 = _k1(sflat, head_p, tail_p, et_p)
    segt = _reduce_hist(seg32, NEPAD)
    entity_agg = _aggregate(entity_emb, head_p, tail_p, p_all, segt,
                            rng=12544, outn=NE, pw=EPAD // NS, c=C2,
                            npass=2)

    pad_n = NNZPAD - NNZ
    user_p = jnp.concatenate([interact_indices[0],
                              jnp.full((pad_n,), USER_PAD, I32)])
    item_p = jnp.concatenate([interact_indices[1], jnp.zeros((pad_n,), I32)])
    val_p = jnp.concatenate([interact_values, jnp.zeros((pad_n,), I32)])

    p2_all, seg32u = _k3(ui, entity_emb, user_p, item_p, val_p)
    segu = _reduce_hist(seg32u, NUPAD)
    user_agg = _aggregate(entity_emb, user_p, item_p, p2_all, segu,
                          rng=10240, outn=NU, pw=NNZPAD // NS, c=C4,
                          npass=1)

    return (entity_agg, user_agg)


# speedup vs baseline: 4.6946x; 1.0931x over previous
import functools

import jax
import jax.numpy as jnp
from jax import lax
from jax.experimental import pallas as pl
from jax.experimental.pallas import tpu as pltpu
from jax.experimental.pallas import tpu_sc as plsc

NE = 50000
NU = 20000
NR = 20
NI = 5
E = 800000
NNZ = 500000
D = 64

NC = 2
NS = 16
NW = NC * NS

NEPAD = 50176
NUPAD = 20480
EPAD = 819200
NNZPAD = 524288
HEAD_PAD = 50175
USER_PAD = 20479

C1 = 2560
C2 = 512
C3 = 512
C4 = 512

F32 = jnp.float32
I32 = jnp.int32

_SC_PARAMS = pltpu.CompilerParams(needs_layout_passes=False,
                                  use_tc_tiling_on_sc=False)


def _mesh():
    return plsc.VectorSubcoreMesh(core_axis_name="c", subcore_axis_name="s")


def _wid():
    return lax.axis_index("s") * NC + lax.axis_index("c")


def _zero_1d(ref, n):
    z = jnp.zeros((16,), ref.dtype)

    @pl.loop(0, n // 16)
    def _z(i):
        ref[pl.ds(i * 16, 16)] = z


def _s_table_body(e_ref, r_ref, o_ref):
    o_ref[...] = jnp.dot(e_ref[...], r_ref[...],
                         preferred_element_type=F32) * 0.125


def _s_table(entity_emb, relation_t):
    return pl.pallas_call(
        _s_table_body,
        grid=(NE // 1000,),
        in_specs=[
            pl.BlockSpec((1000, D), lambda i: (i, 0)),
            pl.BlockSpec((D, NR), lambda i: (0, 0)),
        ],
        out_specs=pl.BlockSpec((1000, NR), lambda i: (i, 0)),
        out_shape=jax.ShapeDtypeStruct((NE, NR), F32),
    )(entity_emb, relation_t)


def _ui_table_body(u_ref, i_ref, o_ref):
    for t in range(NI):
        o_ref[t] = u_ref[...] * i_ref[t][None, :]


def _ui_table(user_emb, interaction_emb):
    return pl.pallas_call(
        _ui_table_body,
        grid=(NU // 1000,),
        in_specs=[
            pl.BlockSpec((1000, D), lambda i: (i, 0)),
            pl.BlockSpec((NI, D), lambda i: (0, 0)),
        ],
        out_specs=pl.BlockSpec((NI, 1000, D), lambda i: (0, i, 0)),
        out_shape=jax.ShapeDtypeStruct((NI, NU, D), F32),
    )(user_emb, interaction_emb)


def _k1_body(sflat, head, tail, et, p_all, seg32,
             tail_v, et_v, head_v, idx_v, s_v, p_v, part_v, gsem):
    wid = _wid()
    _zero_1d(part_v, NEPAD)
    per_w = EPAD // NW

    @pl.loop(0, per_w // C1)
    def _chunk(c):
        base = wid * per_w + c * C1
        pltpu.sync_copy(tail.at[pl.ds(base, C1)], tail_v)
        pltpu.sync_copy(et.at[pl.ds(base, C1)], et_v)
        pltpu.sync_copy(head.at[pl.ds(base, C1)], head_v)

        @pl.loop(0, C1 // 16)
        def _g(g):
            sl = pl.ds(g * 16, 16)
            idx_v[sl] = tail_v[sl] * NR + et_v[sl] - 1

        cps = [
            pltpu.async_copy(sflat.at[idx_v.at[pl.ds(j * 128, 128)]],
                             s_v.at[pl.ds(j * 128, 128)], gsem)
            for j in range(C1 // 128)
        ]
        for cp in cps:
            cp.wait()

        @pl.loop(0, C1 // 16)
        def _g2(g):
            sl = pl.ds(g * 16, 16)
            p16 = jnp.exp(s_v[sl])
            p_v[sl] = p16
            plsc.addupdate_scatter(part_v, [head_v[sl]], p16)

        pltpu.sync_copy(p_v, p_all.at[pl.ds(base, C1)])

    pltpu.sync_copy(part_v, seg32.at[wid])


def _k1(sflat, head, tail, et):
    f = pl.kernel(
        _k1_body,
        out_type=(
            jax.ShapeDtypeStruct((EPAD,), F32),
            jax.ShapeDtypeStruct((NW, NEPAD), F32),
        ),
        mesh=_mesh(),
        scratch_types=[
            pltpu.VMEM((C1,), I32),
            pltpu.VMEM((C1,), I32),
            pltpu.VMEM((C1,), I32),
            pltpu.VMEM((C1,), I32),
            pltpu.VMEM((C1,), F32),
            pltpu.VMEM((C1,), F32),
            pltpu.VMEM((NEPAD,), F32),
            pltpu.SemaphoreType.DMA,
        ],
        compiler_params=_SC_PARAMS,
    )
    return f(sflat, head, tail, et)


def _red_body(seg32, segt, acc_v, tmp_v, *, np_):
    wid = _wid()
    st = np_ // NW
    off = wid * st
    _zero_1d(acc_v, st)

    @pl.loop(0, NW)
    def _i(i):
        pltpu.sync_copy(seg32.at[i, pl.ds(off, st)], tmp_v)

        @pl.loop(0, st // 16)
        def _g(g):
            sl = pl.ds(g * 16, 16)
            acc_v[sl] = acc_v[sl] + tmp_v[sl]

    pltpu.sync_copy(acc_v, segt.at[pl.ds(off, st)])


def _reduce_hist(seg32, np_):
    st = np_ // NW
    f = pl.kernel(
        functools.partial(_red_body, np_=np_),
        out_type=jax.ShapeDtypeStruct((np_,), F32),
        mesh=_mesh(),
        scratch_types=[
            pltpu.VMEM((st,), F32),
            pltpu.VMEM((st,), F32),
        ],
        compiler_params=_SC_PARAMS,
    )
    return f(seg32)


def _agg_body(tab, dsti, srci, p_all, segt, out,
              seg_v, head_v, tail_v, p_v, w_v, lh_v, rows_v, acc_sp,
              gsem, ssem, *, rng, outn, pw, c, npass):
    sc = lax.axis_index("c")
    tl = lax.axis_index("s")
    nsub = c // 128
    iota = lax.iota(I32, 16)
    zch = -(-rng // (NS * 128))
    stripe = zch * 128

    for pa in range(npass):
        rid = pa * NC + sc
        pltpu.sync_copy(segt.at[pl.ds(rid * rng, rng)], seg_v)

        @pl.loop(0, 512)
        def _z(g):
            r = lax.div(g, 4)
            cc = lax.rem(g, 4) * 16
            plsc.store_scatter(rows_v, [jnp.full((16,), r), cc + iota],
                               jnp.zeros((16,), F32))

        @pl.loop(0, zch)
        def _zc(k):
            off = jnp.minimum(tl * stripe + k * 128, rng - 128)
            pltpu.sync_copy(rows_v.at[pl.ds(0, 128)],
                            acc_sp.at[pl.ds(off, 128)])

        plsc.subcore_barrier()

        @pl.loop(0, pw // c)
        def _chunk(ch):
            base = tl * pw + ch * c
            pltpu.sync_copy(dsti.at[pl.ds(base, c)], head_v)
            pltpu.sync_copy(srci.at[pl.ds(base, c)], tail_v)
            pltpu.sync_copy(p_all.at[pl.ds(base, c)], p_v)

            @pl.loop(0, c // 16)
            def _g(g):
                sl = pl.ds(g * 16, 16)
                lh = head_v[sl] - rid * rng
                inr = (lh >= 0) & (lh < rng)
                lhc = jnp.clip(lh, 0, rng - 1)
                dn = plsc.load_gather(seg_v, [lhc])
                w = p_v[sl] / (dn + 1e-16)
                w_v[sl] = jnp.where(inr, w, jnp.zeros((16,), F32))
                j = lax.div(g, 8)
                pos = lax.rem(g, 8) * 16
                plsc.store_scatter(lh_v, [jnp.full((16,), j), pos + iota],
                                   lhc)

            cps = [
                pltpu.async_copy(tab.at[tail_v.at[pl.ds(j * 128, 128)]],
                                 rows_v.at[pl.ds(j * 128, 128)], gsem)
                for j in range(nsub)
            ]
            for cp in cps:
                cp.wait()

            @pl.loop(0, c // 16)
            def _s(g):
                w16 = w_v[pl.ds(g * 16, 16)]
                for i in range(16):
                    r = g * 16 + i
                    wb = lax.broadcast(w16[i], (16,))
                    for q in range(4):
                        sl2 = pl.ds(q * 16, 16)
                        rows_v[r, sl2] = rows_v[r, sl2] * wb

            sps = [
                pltpu.async_copy(rows_v.at[pl.ds(j * 128, 128)],
                                 acc_sp.at[lh_v.at[j]], ssem, add=True)
                for j in range(nsub)
            ]
            for sp in sps:
                sp.wait()

        plsc.subcore_barrier()
        lim = jnp.minimum(rng - 128, outn - 128 - rid * rng)

        @pl.loop(0, zch)
        def _o(k):
            off = jnp.minimum(tl * stripe + k * 128, lim)
            pltpu.sync_copy(acc_sp.at[pl.ds(off, 128)],
                            out.at[pl.ds(rid * rng + off, 128)])

        if pa + 1 < npass:
            plsc.subcore_barrier()


def _aggregate(tab, dsti, srci, p_all, segt, *, rng, outn, pw, c, npass):
    nsub = c // 128
    f = pl.kernel(
        functools.partial(_agg_body, rng=rng, outn=outn, pw=pw, c=c,
                          npass=npass),
        out_type=jax.ShapeDtypeStruct((outn, D), F32),
        mesh=_mesh(),
        scratch_types=[
            pltpu.VMEM((rng,), F32),
            pltpu.VMEM((c,), I32),
            pltpu.VMEM((c,), I32),
            pltpu.VMEM((c,), F32),
            pltpu.VMEM((c,), F32),
            pltpu.VMEM((nsub, 128), I32),
            pltpu.VMEM((c, D), F32),
            pltpu.VMEM_SHARED((rng, D), F32),
            pltpu.SemaphoreType.DMA,
            pltpu.SemaphoreType.DMA,
        ],
        compiler_params=_SC_PARAMS,
    )
    return f(tab, dsti, srci, p_all, segt)


def _k3_body(ui_tab, ent_tab, user, item, val, p2_all, seg32,
             user_v, item_v, val_v, uix_v, att_v, part_v,
             ui_rows, e_rows, gsem):
    wid = _wid()
    _zero_1d(part_v, NUPAD)
    per_w = NNZPAD // NW
    iota = lax.iota(I32, 16)

    @pl.loop(0, per_w // C3)
    def _chunk(c):
        base = wid * per_w + c * C3
        pltpu.sync_copy(user.at[pl.ds(base, C3)], user_v)
        pltpu.sync_copy(item.at[pl.ds(base, C3)], item_v)
        pltpu.sync_copy(val.at[pl.ds(base, C3)], val_v)

        @pl.loop(0, C3 // 16)
        def _g(g):
            sl = pl.ds(g * 16, 16)
            uix_v[sl] = val_v[sl] * NU + user_v[sl]

        cps = []
        for j in range(C3 // 128):
            cps.append(pltpu.async_copy(
                ui_tab.at[uix_v.at[pl.ds(j * 128, 128)]],
                ui_rows.at[pl.ds(j * 128, 128)], gsem))
            cps.append(pltpu.async_copy(
                ent_tab.at[item_v.at[pl.ds(j * 128, 128)]],
                e_rows.at[pl.ds(j * 128, 128)], gsem))
        for cp in cps:
            cp.wait()

        @pl.loop(0, C3 // 16)
        def _dot(g):
            sl = pl.ds(g * 16, 16)
            ridx = g * 16 + iota
            accs = [jnp.zeros((16,), F32) for _ in range(4)]
            for j in range(16):
                skew = (j + iota) & 15
                for q in range(4):
                    cd = q * 16 + skew
                    a = plsc.load_gather(ui_rows, [ridx, cd])
                    b = plsc.load_gather(e_rows, [ridx, cd])
                    accs[q] = accs[q] + a * b
            p16 = jnp.exp((accs[0] + accs[1]) + (accs[2] + accs[3]))
            att_v[sl] = p16
            plsc.addupdate_scatter(part_v, [user_v[sl]], p16)

        pltpu.sync_copy(att_v, p2_all.at[pl.ds(base, C3)])

    pltpu.sync_copy(part_v, seg32.at[wid])


def _k3(ui_tab, ent_tab, user, item, val):
    f = pl.kernel(
        _k3_body,
        out_type=(
            jax.ShapeDtypeStruct((NNZPAD,), F32),
            jax.ShapeDtypeStruct((NW, NUPAD), F32),
        ),
        mesh=_mesh(),
        scratch_types=[
            pltpu.VMEM((C3,), I32),
            pltpu.VMEM((C3,), I32),
            pltpu.VMEM((C3,), I32),
            pltpu.VMEM((C3,), I32),
            pltpu.VMEM((C3,), F32),
            pltpu.VMEM((NUPAD,), F32),
            pltpu.VMEM((C3, D), F32),
            pltpu.VMEM((C3, D), F32),
            pltpu.SemaphoreType.DMA,
        ],
        compiler_params=_SC_PARAMS,
    )
    return f(ui_tab, ent_tab, user, item, val)


def kernel(entity_emb, user_emb, interaction_emb, relation_emb,
           edge_index, edge_type, interact_indices, interact_values):
    head = edge_index[0]
    tail = edge_index[1]

    S = _s_table(entity_emb, relation_emb.T)
    sflat = S.reshape(NE * NR)
    ui = _ui_table(user_emb, interaction_emb).reshape(NI * NU, D)

    pad_e = EPAD - E
    head_p = jnp.concatenate([head, jnp.full((pad_e,), HEAD_PAD, I32)])
    tail_p = jnp.concatenate([tail, jnp.zeros((pad_e,), I32)])
    et_p = jnp.concatenate([edge_type, jnp.ones((pad_e,), I32)])

    p_all, seg32 = _k1(sflat, head_p, tail_p, et_p)
    segt = _reduce_hist(seg32, NEPAD)
    entity_agg = _aggregate(entity_emb, head_p, tail_p, p_all, segt,
                            rng=12544, outn=NE, pw=EPAD // NS, c=C2,
                            npass=2)

    pad_n = NNZPAD - NNZ
    user_p = jnp.concatenate([interact_indices[0],
                              jnp.full((pad_n,), USER_PAD, I32)])
    item_p = jnp.concatenate([interact_indices[1], jnp.zeros((pad_n,), I32)])
    val_p = jnp.concatenate([interact_values, jnp.zeros((pad_n,), I32)])

    p2_all, seg32u = _k3(ui, entity_emb, user_p, item_p, val_p)
    segu = _reduce_hist(seg32u, NUPAD)
    user_agg = _aggregate(entity_emb, user_p, item_p, p2_all, segu,
                          rng=10240, outn=NU, pw=NNZPAD // NS, c=C4,
                          npass=1)

    return (entity_agg, user_agg)


# speedup vs baseline: 6.7883x; 1.4460x over previous
import functools

import jax
import jax.numpy as jnp
from jax import lax
from jax.experimental import pallas as pl
from jax.experimental.pallas import tpu as pltpu
from jax.experimental.pallas import tpu_sc as plsc

NE = 50000
NU = 20000
NR = 20
NI = 5
E = 800000
NNZ = 500000
D = 64

NC = 2
NS = 16
NW = NC * NS

NEPAD = 50176
NUPAD = 20480
EPAD = 819200
NNZPAD = 524288
HEAD_PAD = 50175
USER_PAD = 20479

C1 = 2560
C2 = 512
C3 = 512
C4 = 512

F32 = jnp.float32
I32 = jnp.int32

_SC_PARAMS = pltpu.CompilerParams(needs_layout_passes=False,
                                  use_tc_tiling_on_sc=False)


def _mesh():
    return plsc.VectorSubcoreMesh(core_axis_name="c", subcore_axis_name="s")


def _wid():
    return lax.axis_index("s") * NC + lax.axis_index("c")


def _zero_1d(ref, n):
    z = jnp.zeros((16,), ref.dtype)

    @pl.loop(0, n // 16)
    def _z(i):
        ref[pl.ds(i * 16, 16)] = z


def _s_table_body(e_ref, r_ref, o_ref):
    o_ref[...] = jnp.dot(e_ref[...], r_ref[...],
                         preferred_element_type=F32) * 0.125


def _s_table(entity_emb, relation_t):
    return pl.pallas_call(
        _s_table_body,
        grid=(NE // 1000,),
        in_specs=[
            pl.BlockSpec((1000, D), lambda i: (i, 0)),
            pl.BlockSpec((D, NR), lambda i: (0, 0)),
        ],
        out_specs=pl.BlockSpec((1000, NR), lambda i: (i, 0)),
        out_shape=jax.ShapeDtypeStruct((NE, NR), F32),
    )(entity_emb, relation_t)


def _ui_table_body(u_ref, i_ref, o_ref):
    for t in range(NI):
        o_ref[t] = u_ref[...] * i_ref[t][None, :]


def _ui_table(user_emb, interaction_emb):
    return pl.pallas_call(
        _ui_table_body,
        grid=(NU // 1000,),
        in_specs=[
            pl.BlockSpec((1000, D), lambda i: (i, 0)),
            pl.BlockSpec((NI, D), lambda i: (0, 0)),
        ],
        out_specs=pl.BlockSpec((NI, 1000, D), lambda i: (0, i, 0)),
        out_shape=jax.ShapeDtypeStruct((NI, NU, D), F32),
    )(user_emb, interaction_emb)


CAPE = EPAD // NW + 128
CAPU = NNZPAD // NW + 128


def _partition_step(r16, offs, gidx16, dst16, p16, rings, nrg, rngsz,
                    dstbase_fn, parts):
    newoffs = []
    for r in range(nrg):
        off = offs[r]
        m = r16 == r
        cnt = plsc.all_reduce_population_count(m)
        rank = plsc.cumsum(m.astype(I32)) - 1
        pos = (off + rank) & 255
        rr = jnp.full((16,), r, I32)
        plsc.store_scatter(rings[0], [rr, pos], gidx16, mask=m)
        plsc.store_scatter(rings[1], [rr, pos], dst16 - r * rngsz, mask=m)
        plsc.store_scatter(rings[2], [rr, pos], p16, mask=m)
        offn = off + cnt
        o0 = off[0]
        on0 = offn[0]
        blk = lax.shift_right_logical(o0, 7) & 1
        gw = lax.shift_right_logical(o0, 7) * 128

        @pl.when(lax.shift_right_logical(on0, 7)
                 > lax.shift_right_logical(o0, 7))
        def _fl():
            s128 = pl.ds(blk * 128, 128)
            db = dstbase_fn(r) + gw
            pltpu.sync_copy(rings[0].at[r, s128], parts[0].at[pl.ds(db, 128)])
            pltpu.sync_copy(rings[1].at[r, s128], parts[1].at[pl.ds(db, 128)])
            pltpu.sync_copy(rings[2].at[r, s128], parts[2].at[pl.ds(db, 128)])

        newoffs.append(offn)
    return newoffs


def _partition_finish(offs, rings, nrg, dstbase_fn, parts, cnt_v, cntb, wid):
    iota = lax.iota(I32, 16)
    cnt16 = jnp.zeros((16,), I32)
    for r in range(nrg):
        off = offs[r]
        o0 = off[0]
        blkend = lax.shift_right_logical(o0 + 127, 7) * 128
        for j in range(8):
            pos = off + (j * 16) + iota
            m = pos < blkend
            plsc.store_scatter(rings[2], [jnp.full((16,), r, I32),
                                          pos & 255],
                               jnp.zeros((16,), F32), mask=m)

        @pl.when((o0 & 127) != 0)
        def _fl2():
            blk = lax.shift_right_logical(o0, 7) & 1
            gw = lax.shift_right_logical(o0, 7) * 128
            s128 = pl.ds(blk * 128, 128)
            db = dstbase_fn(r) + gw
            pltpu.sync_copy(rings[0].at[r, s128], parts[0].at[pl.ds(db, 128)])
            pltpu.sync_copy(rings[1].at[r, s128], parts[1].at[pl.ds(db, 128)])
            pltpu.sync_copy(rings[2].at[r, s128], parts[2].at[pl.ds(db, 128)])

        cnt16 = jnp.where(iota == r,
                          lax.shift_right_logical(o0 + 127, 7), cnt16)
    cnt_v[...] = cnt16
    pltpu.sync_copy(cnt_v, cntb.at[wid])


def _zero_rings(rings, nrg):
    iota = lax.iota(I32, 16)

    @pl.loop(0, nrg * 16)
    def _zr(g):
        rr = jnp.full((16,), lax.div(g, 16), I32)
        pos = lax.rem(g, 16) * 16 + iota
        plsc.store_scatter(rings[0], [rr, pos], jnp.zeros((16,), I32))
        plsc.store_scatter(rings[1], [rr, pos], jnp.zeros((16,), I32))
        plsc.store_scatter(rings[2], [rr, pos], jnp.zeros((16,), F32))


def _k1_body(sflat, head, tail, et, seg32, tailp, lhp, pp, cntb,
             tail_v, et_v, head_v, idx_v, s_v, part_v,
             ring_t, ring_l, ring_p, cnt_v, gsem):
    wid = _wid()
    _zero_1d(part_v, NEPAD)
    rings = (ring_t, ring_l, ring_p)
    parts = (tailp, lhp, pp)
    _zero_rings(rings, 4)
    per_w = EPAD // NW
    rngsz = NEPAD // 4

    def dstbase(r):
        return (wid * 4 + r) * CAPE

    offs0 = tuple(jnp.zeros((16,), I32) for _ in range(4))

    @pl.loop(0, per_w // C1, init_carry=offs0)
    def offs_fin(c, offs):
        base = wid * per_w + c * C1
        pltpu.sync_copy(tail.at[pl.ds(base, C1)], tail_v)
        pltpu.sync_copy(et.at[pl.ds(base, C1)], et_v)
        pltpu.sync_copy(head.at[pl.ds(base, C1)], head_v)

        @pl.loop(0, C1 // 16)
        def _g(g):
            sl = pl.ds(g * 16, 16)
            idx_v[sl] = tail_v[sl] * NR + et_v[sl] - 1

        cps = [
            pltpu.async_copy(sflat.at[idx_v.at[pl.ds(j * 128, 128)]],
                             s_v.at[pl.ds(j * 128, 128)], gsem)
            for j in range(C1 // 128)
        ]
        for cp in cps:
            cp.wait()

        @pl.loop(0, C1 // 16, init_carry=offs)
        def offs2(g, offs_i):
            sl = pl.ds(g * 16, 16)
            h16 = head_v[sl]
            p16 = jnp.exp(s_v[sl])
            plsc.addupdate_scatter(part_v, [h16], p16)
            r16 = ((h16 >= rngsz).astype(I32)
                   + (h16 >= 2 * rngsz).astype(I32)
                   + (h16 >= 3 * rngsz).astype(I32))
            no = _partition_step(r16, tuple(offs_i), tail_v[sl], h16, p16,
                                 rings, 4, rngsz, dstbase, parts)
            return tuple(no)

        return offs2

    _partition_finish(offs_fin, rings, 4, dstbase, parts, cnt_v, cntb, wid)
    pltpu.sync_copy(part_v, seg32.at[wid])


def _k1(sflat, head, tail, et):
    f = pl.kernel(
        _k1_body,
        out_type=(
            jax.ShapeDtypeStruct((NW, NEPAD), F32),
            jax.ShapeDtypeStruct((NW * 4 * CAPE,), I32),
            jax.ShapeDtypeStruct((NW * 4 * CAPE,), I32),
            jax.ShapeDtypeStruct((NW * 4 * CAPE,), F32),
            jax.ShapeDtypeStruct((NW, 16), I32),
        ),
        mesh=_mesh(),
        scratch_types=[
            pltpu.VMEM((C1,), I32),
            pltpu.VMEM((C1,), I32),
            pltpu.VMEM((C1,), I32),
            pltpu.VMEM((C1,), I32),
            pltpu.VMEM((C1,), F32),
            pltpu.VMEM((NEPAD,), F32),
            pltpu.VMEM((4, 256), I32),
            pltpu.VMEM((4, 256), I32),
            pltpu.VMEM((4, 256), F32),
            pltpu.VMEM((16,), I32),
            pltpu.SemaphoreType.DMA,
        ],
        compiler_params=_SC_PARAMS,
    )
    return f(sflat, head, tail, et)


def _red_body(seg32, segt, acc_v, tmp_v, *, np_):
    wid = _wid()
    st = np_ // NW
    off = wid * st
    _zero_1d(acc_v, st)

    @pl.loop(0, NW)
    def _i(i):
        pltpu.sync_copy(seg32.at[i, pl.ds(off, st)], tmp_v)

        @pl.loop(0, st // 16)
        def _g(g):
            sl = pl.ds(g * 16, 16)
            acc_v[sl] = acc_v[sl] + tmp_v[sl]

    pltpu.sync_copy(acc_v, segt.at[pl.ds(off, st)])


def _reduce_hist(seg32, np_):
    st = np_ // NW
    f = pl.kernel(
        functools.partial(_red_body, np_=np_),
        out_type=jax.ShapeDtypeStruct((np_,), F32),
        mesh=_mesh(),
        scratch_types=[
            pltpu.VMEM((st,), F32),
            pltpu.VMEM((st,), F32),
        ],
        compiler_params=_SC_PARAMS,
    )
    return f(seg32)


def _agg_body(tab, tpart, lpart, ppart, cntb, segt, out,
              seg_v, tl_v, ll_v, pp_v, w_v, lh2_v, rows_v, cnt_v, acc_sp,
              gsem, ssem, *, rng, outn, cap, nrg, pbase, npass):
    sc = lax.axis_index("c")
    tl = lax.axis_index("s")
    iota = lax.iota(I32, 16)
    zch = -(-rng // (NS * 128))
    stripe = zch * 128

    def do_group(base, nb):
        n = nb * 128
        pltpu.sync_copy(tpart.at[pl.ds(base, n)], tl_v.at[pl.ds(0, n)])
        pltpu.sync_copy(lpart.at[pl.ds(base, n)], ll_v.at[pl.ds(0, n)])
        pltpu.sync_copy(ppart.at[pl.ds(base, n)], pp_v.at[pl.ds(0, n)])

        @pl.loop(0, nb * 8)
        def _g(g):
            sl = pl.ds(g * 16, 16)
            lh16 = ll_v[sl]
            dn = plsc.load_gather(seg_v, [lh16])
            w_v[sl] = pp_v[sl] / (dn + 1e-16)
            j = lax.div(g, 8)
            pos = lax.rem(g, 8) * 16
            plsc.store_scatter(lh2_v, [jnp.full((16,), j), pos + iota], lh16)

        cps = [
            pltpu.async_copy(tab.at[tl_v.at[pl.ds(j * 128, 128)]],
                             rows_v.at[pl.ds(j * 128, 128)], gsem)
            for j in range(nb)
        ]
        for cp in cps:
            cp.wait()

        @pl.loop(0, nb * 8)
        def _s(g):
            w16 = w_v[pl.ds(g * 16, 16)]
            for i in range(16):
                r = g * 16 + i
                wb = lax.broadcast(w16[i], (16,))
                for q in range(4):
                    sl2 = pl.ds(q * 16, 16)
                    rows_v[r, sl2] = rows_v[r, sl2] * wb

        sps = [
            pltpu.async_copy(rows_v.at[pl.ds(j * 128, 128)],
                             acc_sp.at[lh2_v.at[j]], ssem, add=True)
            for j in range(nb)
        ]
        for sp in sps:
            sp.wait()

    for pa in range(pbase, pbase + npass):
        rid = pa * NC + sc
        pltpu.sync_copy(segt.at[pl.ds(rid * rng, rng)], seg_v)

        @pl.loop(0, 512)
        def _z(g):
            r = lax.div(g, 4)
            cc = lax.rem(g, 4) * 16
            plsc.store_scatter(rows_v, [jnp.full((16,), r), cc + iota],
                               jnp.zeros((16,), F32))

        @pl.loop(0, zch)
        def _zc(k):
            off = jnp.minimum(tl * stripe + k * 128, rng - 128)
            pltpu.sync_copy(rows_v.at[pl.ds(0, 128)],
                            acc_sp.at[pl.ds(off, 128)])

        plsc.subcore_barrier()

        for wi in range(2):
            wsrc = tl * 2 + wi
            pltpu.sync_copy(cntb.at[wsrc], cnt_v)
            cv = cnt_v[...]
            nblk = jnp.sum(jnp.where(iota == rid, cv, jnp.zeros((16,), I32)))
            rbase = (wsrc * nrg + rid) * cap
            nfull = lax.div(nblk, 4)
            rem = lax.rem(nblk, 4)

            @pl.loop(0, nfull)
            def _cf(cc):
                do_group(rbase + cc * 512, 4)

            @pl.loop(0, rem)
            def _cr(j):
                do_group(rbase + nfull * 512 + j * 128, 1)

        plsc.subcore_barrier()
        lim = jnp.minimum(rng - 128, outn - 128 - rid * rng)

        @pl.loop(0, zch)
        def _o(k):
            off = jnp.minimum(tl * stripe + k * 128, lim)
            pltpu.sync_copy(acc_sp.at[pl.ds(off, 128)],
                            out.at[pl.ds(rid * rng + off, 128)])

        if pa + 1 < pbase + npass:
            plsc.subcore_barrier()


def _aggregate(tab, tpart, lpart, ppart, cntb, segt, *, rng, outn, cap, nrg,
               pbase=0, npass=1):
    f = pl.kernel(
        functools.partial(_agg_body, rng=rng, outn=outn, cap=cap, nrg=nrg,
                          pbase=pbase, npass=npass),
        out_type=jax.ShapeDtypeStruct((outn, D), F32),
        mesh=_mesh(),
        scratch_types=[
            pltpu.VMEM((rng,), F32),
            pltpu.VMEM((512,), I32),
            pltpu.VMEM((512,), I32),
            pltpu.VMEM((512,), F32),
            pltpu.VMEM((512,), F32),
            pltpu.VMEM((4, 128), I32),
            pltpu.VMEM((512, D), F32),
            pltpu.VMEM((16,), I32),
            pltpu.VMEM_SHARED((rng, D), F32),
            pltpu.SemaphoreType.DMA,
            pltpu.SemaphoreType.DMA,
        ],
        compiler_params=_SC_PARAMS,
    )
    return f(tab, tpart, lpart, ppart, cntb, segt)


def _k3_body(ui_tab, ent_tab, user, item, val, seg32, itemp, lup, pp2, cntb,
             user_v, item_v, val_v, uix_v, part_v,
             ui_rows, e_rows, ring_t, ring_l, ring_p, cnt_v, gsem):
    wid = _wid()
    _zero_1d(part_v, NUPAD)
    rings = (ring_t, ring_l, ring_p)
    parts = (itemp, lup, pp2)
    _zero_rings(rings, 2)
    per_w = NNZPAD // NW
    rngsz = NUPAD // 2
    iota = lax.iota(I32, 16)

    def dstbase(r):
        return (wid * 2 + r) * CAPU

    offs0 = tuple(jnp.zeros((16,), I32) for _ in range(2))

    @pl.loop(0, per_w // C3, init_carry=offs0)
    def offs_fin(c, offs):
        base = wid * per_w + c * C3
        pltpu.sync_copy(user.at[pl.ds(base, C3)], user_v)
        pltpu.sync_copy(item.at[pl.ds(base, C3)], item_v)
        pltpu.sync_copy(val.at[pl.ds(base, C3)], val_v)

        @pl.loop(0, C3 // 16)
        def _g(g):
            sl = pl.ds(g * 16, 16)
            uix_v[sl] = val_v[sl] * NU + user_v[sl]

        cps = []
        for j in range(C3 // 128):
            cps.append(pltpu.async_copy(
                ui_tab.at[uix_v.at[pl.ds(j * 128, 128)]],
                ui_rows.at[pl.ds(j * 128, 128)], gsem))
            cps.append(pltpu.async_copy(
                ent_tab.at[item_v.at[pl.ds(j * 128, 128)]],
                e_rows.at[pl.ds(j * 128, 128)], gsem))
        for cp in cps:
            cp.wait()

        @pl.loop(0, C3 // 16, init_carry=offs)
        def offs2(g, offs_i):
            sl = pl.ds(g * 16, 16)
            u16 = user_v[sl]
            ridx = g * 16 + iota
            accs = [jnp.zeros((16,), F32) for _ in range(4)]
            for j in range(16):
                skew = (j + iota) & 15
                for q in range(4):
                    cd = q * 16 + skew
                    a = plsc.load_gather(ui_rows, [ridx, cd])
                    b = plsc.load_gather(e_rows, [ridx, cd])
                    accs[q] = accs[q] + a * b
            p16 = jnp.exp((accs[0] + accs[1]) + (accs[2] + accs[3]))
            plsc.addupdate_scatter(part_v, [u16], p16)
            r16 = (u16 >= rngsz).astype(I32)
            no = _partition_step(r16, tuple(offs_i), item_v[sl], u16, p16,
                                 rings, 2, rngsz, dstbase, parts)
            return tuple(no)

        return offs2

    _partition_finish(offs_fin, rings, 2, dstbase, parts, cnt_v, cntb, wid)
    pltpu.sync_copy(part_v, seg32.at[wid])


def _k3(ui_tab, ent_tab, user, item, val):
    f = pl.kernel(
        _k3_body,
        out_type=(
            jax.ShapeDtypeStruct((NW, NUPAD), F32),
            jax.ShapeDtypeStruct((NW * 2 * CAPU,), I32),
            jax.ShapeDtypeStruct((NW * 2 * CAPU,), I32),
            jax.ShapeDtypeStruct((NW * 2 * CAPU,), F32),
            jax.ShapeDtypeStruct((NW, 16), I32),
        ),
        mesh=_mesh(),
        scratch_types=[
            pltpu.VMEM((C3,), I32),
            pltpu.VMEM((C3,), I32),
            pltpu.VMEM((C3,), I32),
            pltpu.VMEM((C3,), I32),
            pltpu.VMEM((NUPAD,), F32),
            pltpu.VMEM((C3, D), F32),
            pltpu.VMEM((C3, D), F32),
            pltpu.VMEM((2, 256), I32),
            pltpu.VMEM((2, 256), I32),
            pltpu.VMEM((2, 256), F32),
            pltpu.VMEM((16,), I32),
            pltpu.SemaphoreType.DMA,
        ],
        compiler_params=_SC_PARAMS,
    )
    return f(ui_tab, ent_tab, user, item, val)


def kernel(entity_emb, user_emb, interaction_emb, relation_emb,
           edge_index, edge_type, interact_indices, interact_values):
    head = edge_index[0]
    tail = edge_index[1]

    S = _s_table(entity_emb, relation_emb.T)
    sflat = S.reshape(NE * NR)
    ui = _ui_table(user_emb, interaction_emb).reshape(NI * NU, D)

    pad_e = EPAD - E
    head_p = jnp.concatenate([head, jnp.full((pad_e,), HEAD_PAD, I32)])
    tail_p = jnp.concatenate([tail, jnp.zeros((pad_e,), I32)])
    et_p = jnp.concatenate([edge_type, jnp.ones((pad_e,), I32)])

    seg32, tailp, lhp, pp, cntb = _k1(sflat, head_p, tail_p, et_p)
    segt = _reduce_hist(seg32, NEPAD)
    ea0 = _aggregate(entity_emb, tailp, lhp, pp, cntb, segt,
                     rng=NEPAD // 4, outn=NE, cap=CAPE, nrg=4,
                     pbase=0, npass=1)
    ea1 = _aggregate(entity_emb, tailp, lhp, pp, cntb, segt,
                     rng=NEPAD // 4, outn=NE, cap=CAPE, nrg=4,
                     pbase=1, npass=1)
    entity_agg = jnp.concatenate([ea0[:25088], ea1[25088:]])

    pad_n = NNZPAD - NNZ
    user_p = jnp.concatenate([interact_indices[0],
                              jnp.full((pad_n,), USER_PAD, I32)])
    item_p = jnp.concatenate([interact_indices[1], jnp.zeros((pad_n,), I32)])
    val_p = jnp.concatenate([interact_values, jnp.zeros((pad_n,), I32)])

    seg32u, itemp, lup, pp2, cntb2 = _k3(ui, entity_emb, user_p, item_p,
                                         val_p)
    segu = _reduce_hist(seg32u, NUPAD)
    user_agg = _aggregate(entity_emb, itemp, lup, pp2, cntb2, segu,
                          rng=NUPAD // 2, outn=NU, cap=CAPU, nrg=2)

    return (entity_agg, user_agg)


# speedup vs baseline: 7.0228x; 1.0345x over previous
import functools

import jax
import jax.numpy as jnp
from jax import lax
from jax.experimental import pallas as pl
from jax.experimental.pallas import tpu as pltpu
from jax.experimental.pallas import tpu_sc as plsc

NE = 50000
NU = 20000
NR = 20
NI = 5
E = 800000
NNZ = 500000
D = 64

NC = 2
NS = 16
NW = NC * NS

NEPAD = 50176
NUPAD = 20480
EPAD = 819200
NNZPAD = 524288
HEAD_PAD = 50175
USER_PAD = 20479

C1 = 2560
C2 = 512
C3 = 512
C4 = 512

F32 = jnp.float32
I32 = jnp.int32

_SC_PARAMS = pltpu.CompilerParams(needs_layout_passes=False,
                                  use_tc_tiling_on_sc=False)


def _mesh():
    return plsc.VectorSubcoreMesh(core_axis_name="c", subcore_axis_name="s")


def _wid():
    return lax.axis_index("s") * NC + lax.axis_index("c")


def _zero_1d(ref, n):
    z = jnp.zeros((16,), ref.dtype)

    @pl.loop(0, n // 16)
    def _z(i):
        ref[pl.ds(i * 16, 16)] = z


def _s_table_body(e_ref, r_ref, o_ref):
    o_ref[...] = jnp.dot(e_ref[...], r_ref[...],
                         preferred_element_type=F32) * 0.125


def _s_table(entity_emb, relation_t):
    return pl.pallas_call(
        _s_table_body,
        grid=(NE // 1000,),
        in_specs=[
            pl.BlockSpec((1000, D), lambda i: (i, 0)),
            pl.BlockSpec((D, NR), lambda i: (0, 0)),
        ],
        out_specs=pl.BlockSpec((1000, NR), lambda i: (i, 0)),
        out_shape=jax.ShapeDtypeStruct((NE, NR), F32),
    )(entity_emb, relation_t)


def _ui_table_body(u_ref, i_ref, o_ref):
    for t in range(NI):
        o_ref[t] = u_ref[...] * i_ref[t][None, :]


def _ui_table(user_emb, interaction_emb):
    return pl.pallas_call(
        _ui_table_body,
        grid=(NU // 1000,),
        in_specs=[
            pl.BlockSpec((1000, D), lambda i: (i, 0)),
            pl.BlockSpec((NI, D), lambda i: (0, 0)),
        ],
        out_specs=pl.BlockSpec((NI, 1000, D), lambda i: (0, i, 0)),
        out_shape=jax.ShapeDtypeStruct((NI, NU, D), F32),
    )(user_emb, interaction_emb)


CAPE = EPAD // NW + 128
CAPU = NNZPAD // NW + 128


def _partition_step(r16, offs, gidx16, dst16, p16, rings, nrg, rngsz,
                    dstbase_fn, parts):
    newoffs = []
    for r in range(nrg):
        off = offs[r]
        m = r16 == r
        cnt = plsc.all_reduce_population_count(m)
        rank = plsc.cumsum(m.astype(I32)) - 1
        pos = (off + rank) & 255
        rr = jnp.full((16,), r, I32)
        plsc.store_scatter(rings[0], [rr, pos], gidx16, mask=m)
        plsc.store_scatter(rings[1], [rr, pos], dst16 - r * rngsz, mask=m)
        plsc.store_scatter(rings[2], [rr, pos], p16, mask=m)
        offn = off + cnt
        o0 = off[0]
        on0 = offn[0]
        blk = lax.shift_right_logical(o0, 7) & 1
        gw = lax.shift_right_logical(o0, 7) * 128

        @pl.when(lax.shift_right_logical(on0, 7)
                 > lax.shift_right_logical(o0, 7))
        def _fl():
            s128 = pl.ds(blk * 128, 128)
            db = dstbase_fn(r) + gw
            pltpu.sync_copy(rings[0].at[r, s128], parts[0].at[pl.ds(db, 128)])
            pltpu.sync_copy(rings[1].at[r, s128], parts[1].at[pl.ds(db, 128)])
            pltpu.sync_copy(rings[2].at[r, s128], parts[2].at[pl.ds(db, 128)])

        newoffs.append(offn)
    return newoffs


def _partition_finish(offs, rings, nrg, dstbase_fn, parts, cnt_v, cntb, wid):
    iota = lax.iota(I32, 16)
    cnt16 = jnp.zeros((16,), I32)
    for r in range(nrg):
        off = offs[r]
        o0 = off[0]
        blkend = lax.shift_right_logical(o0 + 127, 7) * 128
        for j in range(8):
            pos = off + (j * 16) + iota
            m = pos < blkend
            plsc.store_scatter(rings[2], [jnp.full((16,), r, I32),
                                          pos & 255],
                               jnp.zeros((16,), F32), mask=m)

        @pl.when((o0 & 127) != 0)
        def _fl2():
            blk = lax.shift_right_logical(o0, 7) & 1
            gw = lax.shift_right_logical(o0, 7) * 128
            s128 = pl.ds(blk * 128, 128)
            db = dstbase_fn(r) + gw
            pltpu.sync_copy(rings[0].at[r, s128], parts[0].at[pl.ds(db, 128)])
            pltpu.sync_copy(rings[1].at[r, s128], parts[1].at[pl.ds(db, 128)])
            pltpu.sync_copy(rings[2].at[r, s128], parts[2].at[pl.ds(db, 128)])

        cnt16 = jnp.where(iota == r,
                          lax.shift_right_logical(o0 + 127, 7), cnt16)
    cnt_v[...] = cnt16
    pltpu.sync_copy(cnt_v, cntb.at[wid])


def _zero_rings(rings, nrg):
    iota = lax.iota(I32, 16)

    @pl.loop(0, nrg * 16)
    def _zr(g):
        rr = jnp.full((16,), lax.div(g, 16), I32)
        pos = lax.rem(g, 16) * 16 + iota
        plsc.store_scatter(rings[0], [rr, pos], jnp.zeros((16,), I32))
        plsc.store_scatter(rings[1], [rr, pos], jnp.zeros((16,), I32))
        plsc.store_scatter(rings[2], [rr, pos], jnp.zeros((16,), F32))


def _k1_body(sflat, head, tail, et, seg32, tailp, lhp, pp, cntb,
             tail_v, et_v, head_v, idx_v, s_v, part_v,
             ring_t, ring_l, ring_p, cnt_v, gsem):
    wid = _wid()
    _zero_1d(part_v, NEPAD)
    rings = (ring_t, ring_l, ring_p)
    parts = (tailp, lhp, pp)
    _zero_rings(rings, 4)
    per_w = EPAD // NW
    rngsz = NEPAD // 4

    def dstbase(r):
        return (wid * 4 + r) * CAPE

    offs0 = tuple(jnp.zeros((16,), I32) for _ in range(4))

    @pl.loop(0, per_w // C1, init_carry=offs0)
    def offs_fin(c, offs):
        base = wid * per_w + c * C1
        pltpu.sync_copy(tail.at[pl.ds(base, C1)], tail_v)
        pltpu.sync_copy(et.at[pl.ds(base, C1)], et_v)
        pltpu.sync_copy(head.at[pl.ds(base, C1)], head_v)

        @pl.loop(0, C1 // 16)
        def _g(g):
            sl = pl.ds(g * 16, 16)
            idx_v[sl] = tail_v[sl] * NR + et_v[sl] - 1

        cps = [
            pltpu.async_copy(sflat.at[idx_v.at[pl.ds(j * 128, 128)]],
                             s_v.at[pl.ds(j * 128, 128)], gsem)
            for j in range(C1 // 128)
        ]
        for cp in cps:
            cp.wait()

        @pl.loop(0, C1 // 16, init_carry=offs)
        def offs2(g, offs_i):
            sl = pl.ds(g * 16, 16)
            h16 = head_v[sl]
            p16 = jnp.exp(s_v[sl])
            plsc.addupdate_scatter(part_v, [h16], p16)
            r16 = ((h16 >= rngsz).astype(I32)
                   + (h16 >= 2 * rngsz).astype(I32)
                   + (h16 >= 3 * rngsz).astype(I32))
            no = _partition_step(r16, tuple(offs_i), tail_v[sl], h16, p16,
                                 rings, 4, rngsz, dstbase, parts)
            return tuple(no)

        return offs2

    _partition_finish(offs_fin, rings, 4, dstbase, parts, cnt_v, cntb, wid)
    pltpu.sync_copy(part_v, seg32.at[wid])


def _k1(sflat, head, tail, et):
    f = pl.kernel(
        _k1_body,
        out_type=(
            jax.ShapeDtypeStruct((NW, NEPAD), F32),
            jax.ShapeDtypeStruct((NW * 4 * CAPE,), I32),
            jax.ShapeDtypeStruct((NW * 4 * CAPE,), I32),
            jax.ShapeDtypeStruct((NW * 4 * CAPE,), F32),
            jax.ShapeDtypeStruct((NW, 16), I32),
        ),
        mesh=_mesh(),
        scratch_types=[
            pltpu.VMEM((C1,), I32),
            pltpu.VMEM((C1,), I32),
            pltpu.VMEM((C1,), I32),
            pltpu.VMEM((C1,), I32),
            pltpu.VMEM((C1,), F32),
            pltpu.VMEM((NEPAD,), F32),
            pltpu.VMEM((4, 256), I32),
            pltpu.VMEM((4, 256), I32),
            pltpu.VMEM((4, 256), F32),
            pltpu.VMEM((16,), I32),
            pltpu.SemaphoreType.DMA,
        ],
        compiler_params=_SC_PARAMS,
    )
    return f(sflat, head, tail, et)


def _red_body(seg32, segt, acc_v, tmp_v, *, np_):
    wid = _wid()
    st = np_ // NW
    off = wid * st
    _zero_1d(acc_v, st)

    @pl.loop(0, NW)
    def _i(i):
        pltpu.sync_copy(seg32.at[i, pl.ds(off, st)], tmp_v)

        @pl.loop(0, st // 16)
        def _g(g):
            sl = pl.ds(g * 16, 16)
            acc_v[sl] = acc_v[sl] + tmp_v[sl]

    pltpu.sync_copy(acc_v, segt.at[pl.ds(off, st)])


def _reduce_hist(seg32, np_):
    st = np_ // NW
    f = pl.kernel(
        functools.partial(_red_body, np_=np_),
        out_type=jax.ShapeDtypeStruct((np_,), F32),
        mesh=_mesh(),
        scratch_types=[
            pltpu.VMEM((st,), F32),
            pltpu.VMEM((st,), F32),
        ],
        compiler_params=_SC_PARAMS,
    )
    return f(seg32)


def _agg_body(tab, tpart, lpart, ppart, cntb, segt, out,
              seg_v, tl_v, ll_v, pp_v, w_v, lh2_v, rows_v, cnt_v, acc_sp,
              gsem, ssem, *, rng, outn, cap, nrg, pbase, npass, gnb):
    sc = lax.axis_index("c")
    tl = lax.axis_index("s")
    iota = lax.iota(I32, 16)
    zch = -(-rng // (NS * 128))
    stripe = zch * 128

    def do_group(base, nb):
        n = nb * 128
        lds = [
            pltpu.async_copy(tpart.at[pl.ds(base, n)], tl_v.at[pl.ds(0, n)],
                             gsem),
            pltpu.async_copy(lpart.at[pl.ds(base, n)], ll_v.at[pl.ds(0, n)],
                             gsem),
            pltpu.async_copy(ppart.at[pl.ds(base, n)], pp_v.at[pl.ds(0, n)],
                             gsem),
        ]
        for ld in lds:
            ld.wait()

        @pl.loop(0, nb * 8)
        def _g(g):
            sl = pl.ds(g * 16, 16)
            lh16 = ll_v[sl]
            dn = plsc.load_gather(seg_v, [lh16])
            w_v[sl] = pp_v[sl] / (dn + 1e-16)
            j = lax.div(g, 8)
            pos = lax.rem(g, 8) * 16
            plsc.store_scatter(lh2_v, [jnp.full((16,), j), pos + iota], lh16)

        cps = [
            pltpu.async_copy(tab.at[tl_v.at[pl.ds(j * 128, 128)]],
                             rows_v.at[pl.ds(j * 128, 128)], gsem)
            for j in range(nb)
        ]
        for cp in cps:
            cp.wait()

        @pl.loop(0, nb * 8)
        def _s(g):
            w16 = w_v[pl.ds(g * 16, 16)]
            for i in range(16):
                r = g * 16 + i
                wb = lax.broadcast(w16[i], (16,))
                for q in range(4):
                    sl2 = pl.ds(q * 16, 16)
                    rows_v[r, sl2] = rows_v[r, sl2] * wb

        sps = [
            pltpu.async_copy(rows_v.at[pl.ds(j * 128, 128)],
                             acc_sp.at[lh2_v.at[j]], ssem, add=True)
            for j in range(nb)
        ]
        for sp in sps:
            sp.wait()

    for pa in range(pbase, pbase + npass):
        rid = pa * NC + sc
        pltpu.sync_copy(segt.at[pl.ds(rid * rng, rng)], seg_v)

        @pl.loop(0, 512)
        def _z(g):
            r = lax.div(g, 4)
            cc = lax.rem(g, 4) * 16
            plsc.store_scatter(rows_v, [jnp.full((16,), r), cc + iota],
                               jnp.zeros((16,), F32))

        @pl.loop(0, zch)
        def _zc(k):
            off = jnp.minimum(tl * stripe + k * 128, rng - 128)
            pltpu.sync_copy(rows_v.at[pl.ds(0, 128)],
                            acc_sp.at[pl.ds(off, 128)])

        plsc.subcore_barrier()

        for wi in range(2):
            wsrc = tl * 2 + wi
            pltpu.sync_copy(cntb.at[wsrc], cnt_v)
            cv = cnt_v[...]
            nblk = jnp.sum(jnp.where(iota == rid, cv, jnp.zeros((16,), I32)))
            rbase = (wsrc * nrg + rid) * cap
            nfull = lax.div(nblk, gnb)
            rem = lax.rem(nblk, gnb)

            @pl.loop(0, nfull)
            def _cf(cc):
                do_group(rbase + cc * (gnb * 128), gnb)

            @pl.loop(0, rem)
            def _cr(j):
                do_group(rbase + nfull * (gnb * 128) + j * 128, 1)

        plsc.subcore_barrier()
        lim = jnp.minimum(rng - 128, outn - 128 - rid * rng)

        @pl.loop(0, zch)
        def _o(k):
            off = jnp.minimum(tl * stripe + k * 128, lim)
            pltpu.sync_copy(acc_sp.at[pl.ds(off, 128)],
                            out.at[pl.ds(rid * rng + off, 128)])

        if pa + 1 < pbase + npass:
            plsc.subcore_barrier()


def _aggregate(tab, tpart, lpart, ppart, cntb, segt, *, rng, outn, cap, nrg,
               pbase=0, npass=1, gnb=4):
    f = pl.kernel(
        functools.partial(_agg_body, rng=rng, outn=outn, cap=cap, nrg=nrg,
                          pbase=pbase, npass=npass, gnb=gnb),
        out_type=jax.ShapeDtypeStruct((outn, D), F32),
        mesh=_mesh(),
        scratch_types=[
            pltpu.VMEM((rng,), F32),
            pltpu.VMEM((gnb * 128,), I32),
            pltpu.VMEM((gnb * 128,), I32),
            pltpu.VMEM((gnb * 128,), F32),
            pltpu.VMEM((gnb * 128,), F32),
            pltpu.VMEM((gnb, 128), I32),
            pltpu.VMEM((gnb * 128, D), F32),
            pltpu.VMEM((16,), I32),
            pltpu.VMEM_SHARED((rng, D), F32),
            pltpu.SemaphoreType.DMA,
            pltpu.SemaphoreType.DMA,
        ],
        compiler_params=_SC_PARAMS,
    )
    return f(tab, tpart, lpart, ppart, cntb, segt)


def _k3_body(ui_tab, ent_tab, user, item, val, seg32, itemp, lup, pp2, cntb,
             user_v, item_v, val_v, uix_v, part_v,
             ui_rows, e_rows, ring_t, ring_l, ring_p, cnt_v, gsem):
    wid = _wid()
    _zero_1d(part_v, NUPAD)
    rings = (ring_t, ring_l, ring_p)
    parts = (itemp, lup, pp2)
    _zero_rings(rings, 2)
    per_w = NNZPAD // NW
    rngsz = NUPAD // 2
    iota = lax.iota(I32, 16)

    def dstbase(r):
        return (wid * 2 + r) * CAPU

    offs0 = tuple(jnp.zeros((16,), I32) for _ in range(2))

    @pl.loop(0, per_w // C3, init_carry=offs0)
    def offs_fin(c, offs):
        base = wid * per_w + c * C3
        pltpu.sync_copy(user.at[pl.ds(base, C3)], user_v)
        pltpu.sync_copy(item.at[pl.ds(base, C3)], item_v)
        pltpu.sync_copy(val.at[pl.ds(base, C3)], val_v)

        @pl.loop(0, C3 // 16)
        def _g(g):
            sl = pl.ds(g * 16, 16)
            uix_v[sl] = val_v[sl] * NU + user_v[sl]

        cps = []
        for j in range(C3 // 128):
            cps.append(pltpu.async_copy(
                ui_tab.at[uix_v.at[pl.ds(j * 128, 128)]],
                ui_rows.at[pl.ds(j * 128, 128)], gsem))
            cps.append(pltpu.async_copy(
                ent_tab.at[item_v.at[pl.ds(j * 128, 128)]],
                e_rows.at[pl.ds(j * 128, 128)], gsem))
        for cp in cps:
            cp.wait()

        @pl.loop(0, C3 // 16, init_carry=offs)
        def offs2(g, offs_i):
            sl = pl.ds(g * 16, 16)
            u16 = user_v[sl]
            ridx = g * 16 + iota
            accs = [jnp.zeros((16,), F32) for _ in range(4)]
            for j in range(16):
                skew = (j + iota) & 15
                for q in range(4):
                    cd = q * 16 + skew
                    a = plsc.load_gather(ui_rows, [ridx, cd])
                    b = plsc.load_gather(e_rows, [ridx, cd])
                    accs[q] = accs[q] + a * b
            p16 = jnp.exp((accs[0] + accs[1]) + (accs[2] + accs[3]))
            plsc.addupdate_scatter(part_v, [u16], p16)
            r16 = (u16 >= rngsz).astype(I32)
            no = _partition_step(r16, tuple(offs_i), item_v[sl], u16, p16,
                                 rings, 2, rngsz, dstbase, parts)
            return tuple(no)

        return offs2

    _partition_finish(offs_fin, rings, 2, dstbase, parts, cnt_v, cntb, wid)
    pltpu.sync_copy(part_v, seg32.at[wid])


def _k3(ui_tab, ent_tab, user, item, val):
    f = pl.kernel(
        _k3_body,
        out_type=(
            jax.ShapeDtypeStruct((NW, NUPAD), F32),
            jax.ShapeDtypeStruct((NW * 2 * CAPU,), I32),
            jax.ShapeDtypeStruct((NW * 2 * CAPU,), I32),
            jax.ShapeDtypeStruct((NW * 2 * CAPU,), F32),
            jax.ShapeDtypeStruct((NW, 16), I32),
        ),
        mesh=_mesh(),
        scratch_types=[
            pltpu.VMEM((C3,), I32),
            pltpu.VMEM((C3,), I32),
            pltpu.VMEM((C3,), I32),
            pltpu.VMEM((C3,), I32),
            pltpu.VMEM((NUPAD,), F32),
            pltpu.VMEM((C3, D), F32),
            pltpu.VMEM((C3, D), F32),
            pltpu.VMEM((2, 256), I32),
            pltpu.VMEM((2, 256), I32),
            pltpu.VMEM((2, 256), F32),
            pltpu.VMEM((16,), I32),
            pltpu.SemaphoreType.DMA,
        ],
        compiler_params=_SC_PARAMS,
    )
    return f(ui_tab, ent_tab, user, item, val)


def kernel(entity_emb, user_emb, interaction_emb, relation_emb,
           edge_index, edge_type, interact_indices, interact_values):
    head = edge_index[0]
    tail = edge_index[1]

    S = _s_table(entity_emb, relation_emb.T)
    sflat = S.reshape(NE * NR)
    ui = _ui_table(user_emb, interaction_emb).reshape(NI * NU, D)

    pad_e = EPAD - E
    head_p = jnp.concatenate([head, jnp.full((pad_e,), HEAD_PAD, I32)])
    tail_p = jnp.concatenate([tail, jnp.zeros((pad_e,), I32)])
    et_p = jnp.concatenate([edge_type, jnp.ones((pad_e,), I32)])

    seg32, tailp, lhp, pp, cntb = _k1(sflat, head_p, tail_p, et_p)
    segt = _reduce_hist(seg32, NEPAD)
    ea0 = _aggregate(entity_emb, tailp, lhp, pp, cntb, segt,
                     rng=NEPAD // 4, outn=NE, cap=CAPE, nrg=4,
                     pbase=0, npass=1)
    ea1 = _aggregate(entity_emb, tailp, lhp, pp, cntb, segt,
                     rng=NEPAD // 4, outn=NE, cap=CAPE, nrg=4,
                     pbase=1, npass=1)
    entity_agg = jnp.concatenate([ea0[:25088], ea1[25088:]])

    pad_n = NNZPAD - NNZ
    user_p = jnp.concatenate([interact_indices[0],
                              jnp.full((pad_n,), USER_PAD, I32)])
    item_p = jnp.concatenate([interact_indices[1], jnp.zeros((pad_n,), I32)])
    val_p = jnp.concatenate([interact_values, jnp.zeros((pad_n,), I32)])

    seg32u, itemp, lup, pp2, cntb2 = _k3(ui, entity_emb, user_p, item_p,
                                         val_p)
    segu = _reduce_hist(seg32u, NUPAD)
    user_agg = _aggregate(entity_emb, itemp, lup, pp2, cntb2, segu,
                          rng=NUPAD // 2, outn=NU, cap=CAPU, nrg=2, gnb=8)

    return (entity_agg, user_agg)
